# baseline (speedup 1.0000x reference)
import functools

import jax
import jax.numpy as jnp
from jax import lax
from jax.experimental import pallas as pl
from jax.experimental.pallas import tpu as pltpu
from jax.experimental.pallas import tpu_sc as plsc

_DIM = 32
_K = 8192
_TOKENS = 8192
_TOK_TILE = 256
_GRID = _TOKENS // _TOK_TILE


def _succ(x):
    return lax.bitcast_convert_type(
        lax.bitcast_convert_type(x, jnp.int32) + 1, jnp.float32)


def _pred(x):
    return lax.bitcast_convert_type(
        lax.bitcast_convert_type(x, jnp.int32) - 1, jnp.float32)


def _tc_body(x_ref, cb_ref, w_ref, imp_ref, idx_ref, loss_ref,
             c2_ref, iota_ref):
    i = pl.program_id(0)

    @pl.when(i == 0)
    def _():
        imp = lax.dot_general(
            cb_ref[...], w_ref[...], (((1,), (1,)), ((), ())),
            preferred_element_type=jnp.float32)
        imp_ref[...] = imp
        c2_ref[...] = jnp.sum(imp * imp, axis=1, keepdims=True).reshape(1, _K)
        iota_ref[...] = lax.broadcasted_iota(
            jnp.int32, (1, _K), 1).astype(jnp.float32)
        loss_ref[0, 0] = 0.0

    xl = x_ref[...]
    x2 = jnp.sum(xl * xl, axis=1, keepdims=True)
    s2 = lax.dot_general(xl + xl, imp_ref[...], (((1,), (1,)), ((), ())),
                         preferred_element_type=jnp.float32)
    d2 = (x2 - s2) + c2_ref[...]
    m2u = jnp.min(d2, axis=1, keepdims=True)
    m2 = jnp.maximum(m2u, 0.0)
    u = jnp.sqrt(m2)
    a = u * _succ(u)
    for _ in range(4):
        a = jnp.where(jnp.sqrt(a) > u, _pred(a), a)
    for _ in range(4):
        b = _succ(a)
        a = jnp.where(jnp.sqrt(b) <= u, b, a)
    a = jnp.where(a >= m2u, a, m2u)
    sel = jnp.where(d2 <= a, iota_ref[...], float(_K))
    idx_ref[...] = jnp.min(sel, axis=1, keepdims=True).astype(jnp.int32)
    loss_ref[0, 0] += jnp.sum(m2) * (1.25 / (_TOKENS * _DIM))


def _tc_distance_argmin(x_flat, codebook, W):
    return pl.pallas_call(
        _tc_body,
        grid=(_GRID,),
        in_specs=[
            pl.BlockSpec((_TOK_TILE, _DIM), lambda i: (i, 0)),
            pl.BlockSpec((_K, _DIM), lambda i: (0, 0)),
            pl.BlockSpec((_DIM, _DIM), lambda i: (0, 0)),
        ],
        out_specs=(
            pl.BlockSpec((_K, _DIM), lambda i: (0, 0)),
            pl.BlockSpec((_TOK_TILE, 1), lambda i: (i, 0)),
            pl.BlockSpec((1, 1), lambda i: (0, 0), memory_space=pltpu.SMEM),
        ),
        out_shape=(
            jax.ShapeDtypeStruct((_K, _DIM), jnp.float32),
            jax.ShapeDtypeStruct((_TOKENS, 1), jnp.int32),
            jax.ShapeDtypeStruct((1, 1), jnp.float32),
        ),
        scratch_shapes=[
            pltpu.VMEM((1, _K), jnp.float32),
            pltpu.VMEM((1, _K), jnp.float32),
        ],
    )(x_flat, codebook, W)


def _sc_gather(implicit, idx_flat):
    info = plsc.get_sparse_core_info()
    nw = info.num_cores * info.num_subcores
    bpw = _TOKENS // nw
    mesh = plsc.VectorSubcoreMesh(core_axis_name="c", subcore_axis_name="s")

    @functools.partial(
        pl.kernel, mesh=mesh,
        compiler_params=pltpu.CompilerParams(use_tc_tiling_on_sc=False),
        out_type=jax.ShapeDtypeStruct((_TOKENS, _DIM), jnp.float32),
        scratch_types=[
            pltpu.VMEM((bpw,), jnp.int32),
            pltpu.VMEM((bpw, _DIM), jnp.float32),
            pltpu.SemaphoreType.DMA,
        ],
    )
    def k(table_hbm, idx_hbm, out_hbm, idx_v, rows_v, sem):
        wid = lax.axis_index("s") * info.num_cores + lax.axis_index("c")
        base = wid * bpw
        pltpu.sync_copy(idx_hbm.at[pl.ds(base, bpw)], idx_v)
        pltpu.async_copy(table_hbm.at[idx_v], rows_v, sem).wait()
        pltpu.sync_copy(rows_v, out_hbm.at[pl.ds(base, bpw)])

    return k(implicit, idx_flat)


def kernel(x, codebook, W):
    B, N, D = x.shape
    x_flat = x.reshape(B * N, D)
    implicit, idx2, loss = _tc_distance_argmin(x_flat, codebook, W)
    idx_flat = idx2.reshape(B * N)
    q_flat = _sc_gather(implicit, idx_flat)
    quantized = q_flat.reshape(B, N, D)
    indices = idx2.reshape(B, N)
    commit_loss = loss[0, 0]
    return quantized, indices, commit_loss


# speedup vs baseline: 1.0181x; 1.0181x over previous
import functools

import jax
import jax.numpy as jnp
from jax import lax
from jax.experimental import pallas as pl
from jax.experimental.pallas import tpu as pltpu
from jax.experimental.pallas import tpu_sc as plsc

_DIM = 32
_K = 8192
_TOKENS = 8192
_TOK_TILE = 512
_GRID = _TOKENS // _TOK_TILE


def _succ(x):
    return lax.bitcast_convert_type(
        lax.bitcast_convert_type(x, jnp.int32) + 1, jnp.float32)


def _pred(x):
    return lax.bitcast_convert_type(
        lax.bitcast_convert_type(x, jnp.int32) - 1, jnp.float32)


def _tc_body(x_ref, cb_ref, w_ref, imp_ref, idx_ref, loss_ref,
             c2_ref, iota_ref):
    i = pl.program_id(0)

    @pl.when(i == 0)
    def _():
        imp = lax.dot_general(
            cb_ref[...], w_ref[...], (((1,), (1,)), ((), ())),
            preferred_element_type=jnp.float32)
        imp_ref[...] = imp
        c2_ref[...] = jnp.sum(imp * imp, axis=1, keepdims=True).reshape(1, _K)
        iota_ref[...] = lax.broadcasted_iota(
            jnp.int32, (1, _K), 1).astype(jnp.float32)
        loss_ref[0, 0] = 0.0

    xl = x_ref[...]
    x2 = jnp.sum(xl * xl, axis=1, keepdims=True)
    s2 = lax.dot_general(xl + xl, imp_ref[...], (((1,), (1,)), ((), ())),
                         preferred_element_type=jnp.float32)
    d2 = (x2 - s2) + c2_ref[...]
    m2u = jnp.min(d2, axis=1, keepdims=True)
    m2 = jnp.maximum(m2u, 0.0)
    u = jnp.sqrt(m2)
    a = u * _succ(u)
    for _ in range(4):
        a = jnp.where(jnp.sqrt(a) > u, _pred(a), a)
    for _ in range(4):
        b = _succ(a)
        a = jnp.where(jnp.sqrt(b) <= u, b, a)
    a = jnp.where(a >= m2u, a, m2u)
    sel = jnp.where(d2 <= a, iota_ref[...], float(_K))
    idx_ref[...] = jnp.min(sel, axis=1, keepdims=True).astype(jnp.int32)
    loss_ref[0, 0] += jnp.sum(m2) * (1.25 / (_TOKENS * _DIM))


def _tc_distance_argmin(x_flat, codebook, W):
    return pl.pallas_call(
        _tc_body,
        grid=(_GRID,),
        in_specs=[
            pl.BlockSpec((_TOK_TILE, _DIM), lambda i: (i, 0)),
            pl.BlockSpec((_K, _DIM), lambda i: (0, 0)),
            pl.BlockSpec((_DIM, _DIM), lambda i: (0, 0)),
        ],
        out_specs=(
            pl.BlockSpec((_K, _DIM), lambda i: (0, 0)),
            pl.BlockSpec((_TOK_TILE, 1), lambda i: (i, 0)),
            pl.BlockSpec((1, 1), lambda i: (0, 0), memory_space=pltpu.SMEM),
        ),
        out_shape=(
            jax.ShapeDtypeStruct((_K, _DIM), jnp.float32),
            jax.ShapeDtypeStruct((_TOKENS, 1), jnp.int32),
            jax.ShapeDtypeStruct((1, 1), jnp.float32),
        ),
        scratch_shapes=[
            pltpu.VMEM((1, _K), jnp.float32),
            pltpu.VMEM((1, _K), jnp.float32),
        ],
    )(x_flat, codebook, W)


def _sc_gather(implicit, idx_flat):
    info = plsc.get_sparse_core_info()
    nw = info.num_cores * info.num_subcores
    bpw = _TOKENS // nw
    mesh = plsc.VectorSubcoreMesh(core_axis_name="c", subcore_axis_name="s")

    @functools.partial(
        pl.kernel, mesh=mesh,
        compiler_params=pltpu.CompilerParams(use_tc_tiling_on_sc=False),
        out_type=jax.ShapeDtypeStruct((_TOKENS, _DIM), jnp.float32),
        scratch_types=[
            pltpu.VMEM((bpw,), jnp.int32),
            pltpu.VMEM((bpw, _DIM), jnp.float32),
            pltpu.SemaphoreType.DMA,
        ],
    )
    def k(table_hbm, idx_hbm, out_hbm, idx_v, rows_v, sem):
        wid = lax.axis_index("s") * info.num_cores + lax.axis_index("c")
        base = wid * bpw
        pltpu.sync_copy(idx_hbm.at[pl.ds(base, bpw)], idx_v)
        pltpu.async_copy(table_hbm.at[idx_v], rows_v, sem).wait()
        pltpu.sync_copy(rows_v, out_hbm.at[pl.ds(base, bpw)])

    return k(implicit, idx_flat)


def kernel(x, codebook, W):
    B, N, D = x.shape
    x_flat = x.reshape(B * N, D)
    implicit, idx2, loss = _tc_distance_argmin(x_flat, codebook, W)
    idx_flat = idx2.reshape(B * N)
    q_flat = _sc_gather(implicit, idx_flat)
    quantized = q_flat.reshape(B, N, D)
    indices = idx2.reshape(B, N)
    commit_loss = loss[0, 0]
    return quantized, indices, commit_loss


# speedup vs baseline: 1.0336x; 1.0152x over previous
import functools

import jax
import jax.numpy as jnp
from jax import lax
from jax.experimental import pallas as pl
from jax.experimental.pallas import tpu as pltpu
from jax.experimental.pallas import tpu_sc as plsc

_DIM = 32
_K = 8192
_TOKENS = 8192
_TOK_TILE = 1024
_GRID = _TOKENS // _TOK_TILE


def _succ(x):
    return lax.bitcast_convert_type(
        lax.bitcast_convert_type(x, jnp.int32) + 1, jnp.float32)


def _pred(x):
    return lax.bitcast_convert_type(
        lax.bitcast_convert_type(x, jnp.int32) - 1, jnp.float32)


def _tc_body(x_ref, cb_ref, w_ref, imp_ref, idx_ref, loss_ref,
             c2_ref, iota_ref):
    i = pl.program_id(0)

    @pl.when(i == 0)
    def _():
        imp = lax.dot_general(
            cb_ref[...], w_ref[...], (((1,), (1,)), ((), ())),
            preferred_element_type=jnp.float32)
        imp_ref[...] = imp
        c2_ref[...] = jnp.sum(imp * imp, axis=1, keepdims=True).reshape(1, _K)
        iota_ref[...] = lax.broadcasted_iota(
            jnp.int32, (1, _K), 1).astype(jnp.float32)
        loss_ref[0, 0] = 0.0

    xl = x_ref[...]
    x2 = jnp.sum(xl * xl, axis=1, keepdims=True)
    s2 = lax.dot_general(xl + xl, imp_ref[...], (((1,), (1,)), ((), ())),
                         preferred_element_type=jnp.float32)
    d2 = (x2 - s2) + c2_ref[...]
    m2u = jnp.min(d2, axis=1, keepdims=True)
    m2 = jnp.maximum(m2u, 0.0)
    u = jnp.sqrt(m2)
    a = u * _succ(u)
    for _ in range(4):
        a = jnp.where(jnp.sqrt(a) > u, _pred(a), a)
    for _ in range(4):
        b = _succ(a)
        a = jnp.where(jnp.sqrt(b) <= u, b, a)
    a = jnp.where(a >= m2u, a, m2u)
    sel = jnp.where(d2 <= a, iota_ref[...], float(_K))
    idx_ref[...] = jnp.min(sel, axis=1, keepdims=True).astype(jnp.int32)
    loss_ref[0, 0] += jnp.sum(m2) * (1.25 / (_TOKENS * _DIM))


def _tc_distance_argmin(x_flat, codebook, W):
    return pl.pallas_call(
        _tc_body,
        grid=(_GRID,),
        in_specs=[
            pl.BlockSpec((_TOK_TILE, _DIM), lambda i: (i, 0)),
            pl.BlockSpec((_K, _DIM), lambda i: (0, 0)),
            pl.BlockSpec((_DIM, _DIM), lambda i: (0, 0)),
        ],
        out_specs=(
            pl.BlockSpec((_K, _DIM), lambda i: (0, 0)),
            pl.BlockSpec((_TOK_TILE, 1), lambda i: (i, 0)),
            pl.BlockSpec((1, 1), lambda i: (0, 0), memory_space=pltpu.SMEM),
        ),
        out_shape=(
            jax.ShapeDtypeStruct((_K, _DIM), jnp.float32),
            jax.ShapeDtypeStruct((_TOKENS, 1), jnp.int32),
            jax.ShapeDtypeStruct((1, 1), jnp.float32),
        ),
        scratch_shapes=[
            pltpu.VMEM((1, _K), jnp.float32),
            pltpu.VMEM((1, _K), jnp.float32),
        ],
    )(x_flat, codebook, W)


def _sc_gather(implicit, idx_flat):
    info = plsc.get_sparse_core_info()
    nw = info.num_cores * info.num_subcores
    bpw = _TOKENS // nw
    mesh = plsc.VectorSubcoreMesh(core_axis_name="c", subcore_axis_name="s")

    @functools.partial(
        pl.kernel, mesh=mesh,
        compiler_params=pltpu.CompilerParams(use_tc_tiling_on_sc=False),
        out_type=jax.ShapeDtypeStruct((_TOKENS, _DIM), jnp.float32),
        scratch_types=[
            pltpu.VMEM((bpw,), jnp.int32),
            pltpu.VMEM((bpw, _DIM), jnp.float32),
            pltpu.SemaphoreType.DMA,
        ],
    )
    def k(table_hbm, idx_hbm, out_hbm, idx_v, rows_v, sem):
        wid = lax.axis_index("s") * info.num_cores + lax.axis_index("c")
        base = wid * bpw
        pltpu.sync_copy(idx_hbm.at[pl.ds(base, bpw)], idx_v)
        pltpu.async_copy(table_hbm.at[idx_v], rows_v, sem).wait()
        pltpu.sync_copy(rows_v, out_hbm.at[pl.ds(base, bpw)])

    return k(implicit, idx_flat)


def kernel(x, codebook, W):
    B, N, D = x.shape
    x_flat = x.reshape(B * N, D)
    implicit, idx2, loss = _tc_distance_argmin(x_flat, codebook, W)
    idx_flat = idx2.reshape(B * N)
    q_flat = _sc_gather(implicit, idx_flat)
    quantized = q_flat.reshape(B, N, D)
    indices = idx2.reshape(B, N)
    commit_loss = loss[0, 0]
    return quantized, indices, commit_loss
